# baseline (speedup 1.0000x reference)
import jax
import jax.numpy as jnp
from jax.experimental import pallas as pl
from jax.experimental.pallas import tpu as pltpu

_B, _A, _R, _C, _CH = 32, 3, 52, 52, 85
_RB = 26


def _body(anch_ref, pred_ref, targ_ref, box_ref, loss_ref, acc_ref):
    a = pl.program_id(0)
    j = pl.program_id(1)

    @pl.when((a == 0) & (j == 0))
    def _init():
        acc_ref[0] = 0.0
        acc_ref[1] = 0.0

    w = anch_ref[a, 0]
    h = anch_ref[a, 1]
    s_bce = 0.0
    s_cnt = 0.0
    for k in range(_RB):
        p = jnp.transpose(pred_ref[0, k, :, :, 0:8], (2, 1, 0))

        x = p[4:5]
        t = targ_ref[0, k, 0, :, :][None]
        bce = jnp.maximum(x, 0.0) + jnp.log1p(jnp.exp(-jnp.abs(x)))
        mask = t == 0.0
        s_bce += jnp.sum(jnp.where(mask, bce, 0.0))
        s_cnt += jnp.sum(mask.astype(jnp.float32))

        s = 1.0 / (1.0 + jnp.exp(-p[0:2]))
        row = j * _RB + k
        r2 = row // 2
        half = 4 * (row % 2)
        box_ref[:, 0, r2, half + 0, :] = s[0]
        box_ref[:, 0, r2, half + 1, :] = s[1]
        box_ref[:, 0, r2, half + 2, :] = jnp.exp(p[2]) * w
        box_ref[:, 0, r2, half + 3, :] = jnp.exp(p[3]) * h

    acc_ref[0] += s_bce
    acc_ref[1] += s_cnt

    @pl.when((a == _A - 1) & (j == _R // _RB - 1))
    def _fin():
        loss_ref[0] = acc_ref[0] / acc_ref[1]


def kernel(pred, target, scaled_anchors):
    pred_t = jnp.transpose(pred, (1, 2, 3, 0, 4))
    targ_t = jnp.transpose(target, (1, 2, 4, 0, 3))

    box_q, loss = pl.pallas_call(
        _body,
        grid=(_A, _R // _RB),
        in_specs=[
            pl.BlockSpec(memory_space=pltpu.SMEM),
            pl.BlockSpec((1, _RB, _C, _B, _CH), lambda a, j: (a, j, 0, 0, 0)),
            pl.BlockSpec((1, _RB, 1, _B, _C), lambda a, j: (a, j, 4, 0, 0)),
        ],
        out_specs=[
            pl.BlockSpec((_B, 1, _R // 2, 8, _C), lambda a, j: (0, a, 0, 0, 0)),
            pl.BlockSpec(memory_space=pltpu.SMEM),
        ],
        out_shape=[
            jax.ShapeDtypeStruct((_B, _A, _R // 2, 8, _C), jnp.float32),
            jax.ShapeDtypeStruct((1,), jnp.float32),
        ],
        scratch_shapes=[pltpu.SMEM((2,), jnp.float32)],
    )(scaled_anchors, pred_t, targ_t)

    box = jnp.transpose(box_q.reshape(_B, _A, _R, 4, _C), (0, 1, 2, 4, 3))
    return loss[0], box


# speedup vs baseline: 1.0185x; 1.0185x over previous
import jax
import jax.numpy as jnp
from jax.experimental import pallas as pl
from jax.experimental.pallas import tpu as pltpu

_B, _A, _R, _C, _CH = 32, 3, 52, 52, 85
_RB = 13


def _body(anch_ref, predA_ref, predB_ref, targ_ref, box_ref, loss_ref, acc_ref):
    a = pl.program_id(0)
    j = pl.program_id(1)

    @pl.when((a == 0) & (j == 0))
    def _init():
        acc_ref[0] = 0.0
        acc_ref[1] = 0.0

    w = anch_ref[a, 0]
    h = anch_ref[a, 1]
    s_bce = 0.0
    s_cnt = 0.0
    for half_ref, off in ((predA_ref, 0), (predB_ref, _RB)):
        for k in range(_RB):
            p = jnp.transpose(half_ref[0, k, :, :, 0:8], (2, 1, 0))

            x = p[4:5]
            t = targ_ref[0, off + k, 0, :, :][None]
            bce = jnp.maximum(x, 0.0) + jnp.log1p(jnp.exp(-jnp.abs(x)))
            mask = t == 0.0
            s_bce += jnp.sum(jnp.where(mask, bce, 0.0))
            s_cnt += jnp.sum(mask.astype(jnp.float32))

            s = 1.0 / (1.0 + jnp.exp(-p[0:2]))
            row = off + k
            r2 = row // 2
            hf = 4 * (row % 2)
            box_ref[:, 0, r2, hf + 0, :] = s[0]
            box_ref[:, 0, r2, hf + 1, :] = s[1]
            box_ref[:, 0, r2, hf + 2, :] = jnp.exp(p[2]) * w
            box_ref[:, 0, r2, hf + 3, :] = jnp.exp(p[3]) * h

    acc_ref[0] += s_bce
    acc_ref[1] += s_cnt

    @pl.when((a == _A - 1) & (j == _R // (2 * _RB) - 1))
    def _fin():
        loss_ref[0] = acc_ref[0] / acc_ref[1]


def kernel(pred, target, scaled_anchors):
    pred_t = jnp.transpose(pred, (1, 2, 3, 0, 4))
    targ_t = jnp.transpose(target, (1, 2, 4, 0, 3))

    box_q, loss = pl.pallas_call(
        _body,
        grid=(_A, _R // (2 * _RB)),
        in_specs=[
            pl.BlockSpec(memory_space=pltpu.SMEM),
            pl.BlockSpec((1, _RB, _C, _B, _CH), lambda a, j: (a, 2 * j, 0, 0, 0)),
            pl.BlockSpec((1, _RB, _C, _B, _CH), lambda a, j: (a, 2 * j + 1, 0, 0, 0)),
            pl.BlockSpec((1, 2 * _RB, 1, _B, _C), lambda a, j: (a, j, 4, 0, 0)),
        ],
        out_specs=[
            pl.BlockSpec((_B, 1, _RB, 8, _C), lambda a, j: (0, a, j, 0, 0)),
            pl.BlockSpec(memory_space=pltpu.SMEM),
        ],
        out_shape=[
            jax.ShapeDtypeStruct((_B, _A, _R // 2, 8, _C), jnp.float32),
            jax.ShapeDtypeStruct((1,), jnp.float32),
        ],
        scratch_shapes=[pltpu.SMEM((2,), jnp.float32)],
    )(scaled_anchors, pred_t, pred_t, targ_t)

    box = jnp.transpose(box_q.reshape(_B, _A, _R, 4, _C), (0, 1, 2, 4, 3))
    return loss[0], box
